# baseline (device time: 73814 ns/iter reference)
import jax
import jax.numpy as jnp
from jax import lax
from jax.experimental import pallas as pl
from jax.experimental.pallas import tpu as pltpu

N_DEV = 32
NZ = 4
NP = 8

_R_OF_P = (0, 1, 2, 7, 6, 3, 4, 5)
_NEXT_P = (1, 2, 5, 0, 3, 6, 7, 4)
_PREV_P = (3, 0, 1, 4, 7, 2, 5, 6)


def _lut(table, idx):
    res = jnp.int32(table[0])
    for i in range(1, len(table)):
        res = jnp.where(idx == i, jnp.int32(table[i]), res)
    return res


def _gelu(z):
    return 0.5 * z * (1.0 + jnp.tanh(0.7978845608 * (z + 0.044715 * z * z * z)))


def kernel(A, B):
    m, k = A.shape
    k2, n = B.shape
    assert k == k2
    half = m // 2
    prow = half // NP
    NS = 4
    hrow = prow // NS
    zrow = prow // NZ
    zslab = 2 * zrow

    def body(a_ref, b_ref, out_ref, pcp, pcm, zw, zc, psp_s, psp_r, psm_s,
             psm_r, zs_s, zs_r):
        my = lax.axis_index("i")
        z = my // NP
        p = my % NP
        r = _lut(_R_OF_P, p)
        z_left = (my - NP) % N_DEV
        z_right = (my + NP) % N_DEV
        p_right = z * NP + _lut(_NEXT_P, p)
        p_left = z * NP + _lut(_PREV_P, p)

        barrier_sem = pltpu.get_barrier_semaphore()
        for nbr in (z_left, z_right, p_left, p_right):
            pl.semaphore_signal(
                barrier_sem, inc=1,
                device_id=(nbr,), device_id_type=pl.DeviceIdType.MESH,
            )
        pl.semaphore_wait(barrier_sem, 4)

        def row_p(c):
            return c * prow

        def row_m(c):
            return half + c * prow

        def mm_block(row):
            out_ref[pl.ds(row, prow), :] = jnp.dot(
                a_ref[pl.ds(row, prow), :], b_ref[:, :],
                preferred_element_type=jnp.float32,
            )

        mm_block(row_p(r % NP))
        mm_block(row_m(r % NP))
        mm_block(row_p((r - 1) % NP))
        mm_block(row_m((r + 1) % NP))

        def mk_plus(s, chunk, h):
            return pltpu.make_async_remote_copy(
                src_ref=out_ref.at[pl.ds(row_p(chunk) + h * hrow, hrow), :],
                dst_ref=pcp.at[s, pl.ds(h * hrow, hrow)],
                send_sem=psp_s.at[s, h], recv_sem=psp_r.at[s, h],
                device_id=(p_right,), device_id_type=pl.DeviceIdType.MESH,
            )

        def mk_minus(s, chunk, h):
            return pltpu.make_async_remote_copy(
                src_ref=out_ref.at[pl.ds(row_m(chunk) + h * hrow, hrow), :],
                dst_ref=pcm.at[s, pl.ds(h * hrow, hrow)],
                send_sem=psm_s.at[s, h], recv_sem=psm_r.at[s, h],
                device_id=(p_left,), device_id_type=pl.DeviceIdType.MESH,
            )

        rs_sends = []
        st = []
        for h in range(NS):
            st += [mk_plus(0, r % NP, h), mk_minus(0, r % NP, h)]
        for rd in st:
            rd.start()
        for s in range(NP - 1):
            rc_p = (r - s - 1) % NP
            rc_m = (r + s + 1) % NP
            nxt = []
            for idx, rd in enumerate(st):
                plus = (idx % 2 == 0)
                h = idx // 2
                rd.wait_recv()
                if plus:
                    out_ref[pl.ds(row_p(rc_p) + h * hrow, hrow), :] += (
                        pcp[s, h * hrow:(h + 1) * hrow]
                    )
                else:
                    out_ref[pl.ds(row_m(rc_m) + h * hrow, hrow), :] += (
                        pcm[s, h * hrow:(h + 1) * hrow]
                    )
                rs_sends.append(rd)
                if s + 1 < NP - 1:
                    nrd = (mk_plus(s + 1, rc_p, h) if plus
                           else mk_minus(s + 1, rc_m, h))
                    nrd.start()
                    nxt.append(nrd)
            st = nxt
            if s < NP - 2:
                mm_block(row_p((r - s - 2) % NP))
                mm_block(row_m((r + s + 2) % NP))

        c_l = (r + 1) % NP
        c_r = (r - 1) % NP
        base_l = c_l * prow
        base_m = half + c_r * prow

        for j in range(NZ):
            zw[j, :zrow, :] = out_ref[pl.ds(base_l + j * zrow, zrow), :]
            zw[j, zrow:, :] = out_ref[pl.ds(base_m + j * zrow, zrow), :]

        def mk_z(s, chunk, h):
            return pltpu.make_async_remote_copy(
                src_ref=zw.at[chunk, pl.ds(h * zrow, zrow)],
                dst_ref=zc.at[s, pl.ds(h * zrow, zrow)],
                send_sem=zs_s.at[s, h], recv_sem=zs_r.at[s, h],
                device_id=(z_right,), device_id_type=pl.DeviceIdType.MESH,
            )

        z_sends = []
        zst = [mk_z(0, z % NZ, 0), mk_z(0, z % NZ, 1)]
        for rd in zst:
            rd.start()
        for s in range(NZ - 1):
            rzc = (z - s - 1) % NZ
            nxt = []
            for h, rd in enumerate(zst):
                rd.wait_recv()
                zw[rzc, h * zrow:(h + 1) * zrow, :] += (
                    zc[s, h * zrow:(h + 1) * zrow]
                )
                z_sends.append(rd)
                if s + 1 < NZ - 1:
                    nrd = mk_z(s + 1, rzc, h)
                    nrd.start()
                    nxt.append(nrd)
            zst = nxt

        zc_own = (z + 1) % NZ
        zw[zc_own, :, :] = _gelu(zw[zc_own, :, :])

        for rd in z_sends:
            rd.wait_send()

        zag_sends = []
        zst = [mk_z(NZ - 1, zc_own, 0), mk_z(NZ - 1, zc_own, 1)]
        for rd in zst:
            rd.start()
        for s in range(NZ - 1):
            t = (NZ - 1) + s
            rzc = (z - s) % NZ
            nxt = []
            for h, rd in enumerate(zst):
                rd.wait_recv()
                zw[rzc, h * zrow:(h + 1) * zrow, :] = (
                    zc[t, h * zrow:(h + 1) * zrow]
                )
                zag_sends.append(rd)
                if s + 1 < NZ - 1:
                    nrd = mk_z(t + 1, rzc, h)
                    nrd.start()
                    nxt.append(nrd)
            zst = nxt

        for j in range(NZ):
            out_ref[pl.ds(base_l + j * zrow, zrow), :] = zw[j, :zrow, :]
            out_ref[pl.ds(base_m + j * zrow, zrow), :] = zw[j, zrow:, :]

        for rd in rs_sends:
            rd.wait_send()

        ag_sends = []
        st = []
        for h in range(NS):
            st += [mk_plus(NP - 1, c_l, h), mk_minus(NP - 1, c_r, h)]
        for rd in st:
            rd.start()
        for s in range(NP - 1):
            t = (NP - 1) + s
            rc_p = (r - s) % NP
            rc_m = (r + s) % NP
            nxt = []
            for idx, rd in enumerate(st):
                plus = (idx % 2 == 0)
                h = idx // 2
                rd.wait_recv()
                if plus:
                    out_ref[pl.ds(row_p(rc_p) + h * hrow, hrow), :] = (
                        pcp[t, h * hrow:(h + 1) * hrow]
                    )
                else:
                    out_ref[pl.ds(row_m(rc_m) + h * hrow, hrow), :] = (
                        pcm[t, h * hrow:(h + 1) * hrow]
                    )
                ag_sends.append(rd)
                if s + 1 < NP - 1:
                    nrd = (mk_plus(t + 1, rc_p, h) if plus
                           else mk_minus(t + 1, rc_m, h))
                    nrd.start()
                    nxt.append(nrd)
            st = nxt

        for rd in zag_sends + ag_sends:
            rd.wait_send()

    n_p = 2 * (NP - 1)
    n_zs = 2 * (NZ - 1)
    return pl.pallas_call(
        body,
        out_shape=jax.ShapeDtypeStruct((m, n), jnp.float32),
        in_specs=[
            pl.BlockSpec(memory_space=pltpu.VMEM),
            pl.BlockSpec(memory_space=pltpu.VMEM),
        ],
        out_specs=pl.BlockSpec(memory_space=pltpu.VMEM),
        scratch_shapes=[
            pltpu.VMEM((n_p, prow, n), jnp.float32),
            pltpu.VMEM((n_p, prow, n), jnp.float32),
            pltpu.VMEM((NZ, zslab, n), jnp.float32),
            pltpu.VMEM((n_zs, zslab, n), jnp.float32),
            pltpu.SemaphoreType.DMA((n_p, NS)),
            pltpu.SemaphoreType.DMA((n_p, NS)),
            pltpu.SemaphoreType.DMA((n_p, NS)),
            pltpu.SemaphoreType.DMA((n_p, NS)),
            pltpu.SemaphoreType.DMA((n_zs, 2)),
            pltpu.SemaphoreType.DMA((n_zs, 2)),
        ],
        compiler_params=pltpu.CompilerParams(collective_id=0),
    )(A, B)
